# baseline (speedup 1.0000x reference)
import jax
import jax.numpy as jnp
from jax import lax
from jax.experimental import pallas as pl
from jax.experimental.pallas import tpu as pltpu
from jax.experimental.pallas import tpu_sc as plsc

N_NODES = 10000
N_EDGES = 160000
N_REL = 5
N_BASES = 5
D_OUT = 300
D_HALF = 150
D_PAD = 160
RN = N_REL * N_NODES
NSUB = 16
EDGES_PER_SUB = N_EDGES // NSUB
CHUNK = 80
N_ACC = 10112
ROWS_PER_SUB = N_ACC // NSUB
NB_BUILD = 2000
NBLK_BUILD = N_NODES // NB_BUILD
PACK_ROWS = RN * 2 * D_PAD // 128
ROW_VIEW = RN * 2


def _pack160(h):
    nb = h.shape[0]
    h3 = h.reshape(nb // 4, 4, D_PAD)
    q = jnp.stack([
        h3[:, 0, 0:128],
        jnp.concatenate([h3[:, 0, 128:160], h3[:, 1, 0:96]], axis=-1),
        jnp.concatenate([h3[:, 1, 96:160], h3[:, 2, 0:64]], axis=-1),
        jnp.concatenate([h3[:, 2, 64:160], h3[:, 3, 0:32]], axis=-1),
        h3[:, 3, 32:160],
    ], axis=1)
    return q.reshape(nb * D_PAD // 128, 128)


def _build_w_body(comp_ref, et_ref, src_ref, basis_ref, out_ref, eidx_ref):
    r = pl.program_id(1)
    w = comp_ref[r, 0] * basis_ref[0]
    for b in range(1, N_BASES):
        w = w + comp_ref[r, b] * basis_ref[b]
    nb = w.shape[0]
    col = lax.broadcasted_iota(jnp.int32, (nb, D_PAD), 1)
    h0 = jnp.where(col < D_HALF, w[:, :D_PAD],
                   jnp.where(col == D_HALF, 1.0, 0.0))
    h1 = w[:, D_OUT - D_PAD:]
    out_ref[...] = jnp.concatenate([_pack160(h0), _pack160(h1)], axis=0)
    base = (et_ref[...] * NBLK_BUILD + src_ref[...] // NB_BUILD) * \
        (2 * NB_BUILD) + src_ref[...] % NB_BUILD
    eidx_ref[0] = base
    eidx_ref[1] = base + NB_BUILD


def _build_w(comp, et2, src2, basis, nb=NB_BUILD):
    nblk = N_NODES // nb
    return pl.pallas_call(
        _build_w_body,
        grid=(nblk, N_REL),
        in_specs=[
            pl.BlockSpec(memory_space=pltpu.SMEM),
            pl.BlockSpec((16, 2000), lambda n, r: (n, 0)),
            pl.BlockSpec((16, 2000), lambda n, r: (n, 0)),
            pl.BlockSpec((N_BASES, nb, D_OUT), lambda n, r: (0, n, 0)),
        ],
        out_specs=[
            pl.BlockSpec((2 * nb * D_PAD // 128, 128),
                         lambda n, r: (r * nblk + n, 0)),
            pl.BlockSpec((2, 16, 2000), lambda n, r: (0, n, 0)),
        ],
        out_shape=[
            jax.ShapeDtypeStruct((PACK_ROWS, 128), jnp.float32),
            jax.ShapeDtypeStruct((2, 80, 2000), jnp.int32),
        ],
    )(comp, et2, src2, basis)


EBLK = 2000
NBLK_E = EDGES_PER_SUB // EBLK
CH_PER_BLK = EBLK // CHUNK


def _sc_agg_body(eidx_hbm, dst_hbm, w_hbm, out_hbm,
                 idx_v, dst_v, rows_v, acc_sh, sem):
    c = lax.axis_index("c")
    s = lax.axis_index("s")

    def zrow(i, carry):
        def zcol(k, carry2):
            rows_v[i // CHUNK, i % CHUNK, pl.ds(k * 16, 16)] = (
                jnp.zeros((16,), jnp.float32))
            return carry2
        return lax.fori_loop(0, D_PAD // 16, zcol, carry)
    lax.fori_loop(0, 2 * CHUNK, zrow, 0)

    row0 = s * ROWS_PER_SUB
    nfull = ROWS_PER_SUB // CHUNK
    tail = ROWS_PER_SUB - nfull * CHUNK

    def zcopy(q, carry):
        pltpu.sync_copy(rows_v.at[0], acc_sh.at[pl.ds(row0 + q * CHUNK, CHUNK)])
        return carry
    lax.fori_loop(0, nfull, zcopy, 0)
    pltpu.sync_copy(rows_v.at[0, pl.ds(0, tail)],
                    acc_sh.at[pl.ds(row0 + nfull * CHUNK, tail)])
    plsc.subcore_barrier()

    w_half = w_hbm

    def block_step(k, carry):
        pltpu.sync_copy(eidx_hbm.at[c, s, pl.ds(k * CH_PER_BLK, CH_PER_BLK)],
                        idx_v)
        pltpu.sync_copy(dst_hbm.at[s, pl.ds(k * CH_PER_BLK, CH_PER_BLK)], dst_v)

        pltpu.async_copy(w_half.at[idx_v.at[0]], rows_v.at[0], sem)

        def chunk_step(j, carry2):
            b = lax.rem(j, 2)
            pltpu.make_async_copy(w_half.at[idx_v.at[0]], rows_v.at[b], sem).wait()
            pltpu.async_copy(w_half.at[idx_v.at[j + 1]], rows_v.at[1 - b], sem)
            pltpu.sync_copy(rows_v.at[b], acc_sh.at[dst_v.at[j]], add=True)
            return carry2
        lax.fori_loop(0, CH_PER_BLK - 1, chunk_step, 0)
        bl = lax.rem(CH_PER_BLK - 1, 2)
        pltpu.make_async_copy(w_half.at[idx_v.at[0]], rows_v.at[bl], sem).wait()
        pltpu.sync_copy(rows_v.at[bl], acc_sh.at[dst_v.at[CH_PER_BLK - 1]],
                        add=True)
        return carry
    lax.fori_loop(0, NBLK_E, block_step, 0)

    plsc.subcore_barrier()
    pltpu.sync_copy(acc_sh.at[pl.ds(row0, ROWS_PER_SUB)],
                    out_hbm.at[pl.ds(c * N_ACC + row0, ROWS_PER_SUB)])


def _sc_aggregate(eidx3, dst3, w3):
    mesh = plsc.VectorSubcoreMesh(core_axis_name="c", subcore_axis_name="s")
    f = pl.kernel(
        _sc_agg_body,
        mesh=mesh,
        compiler_params=pltpu.CompilerParams(use_tc_tiling_on_sc=False),
        out_type=jax.ShapeDtypeStruct((2 * N_ACC, D_PAD), jnp.float32),
        scratch_types=[
            pltpu.VMEM((CH_PER_BLK, CHUNK), jnp.int32),
            pltpu.VMEM((CH_PER_BLK, CHUNK), jnp.int32),
            pltpu.VMEM((2, CHUNK, D_PAD), jnp.float32),
            pltpu.VMEM_SHARED((N_ACC, D_PAD), jnp.float32),
            pltpu.SemaphoreType.DMA,
        ],
    )
    return f(eidx3, dst3, w3)


def _final_body(p0_ref, p1_ref, root_ref, bias_ref, out_ref):
    p0 = p0_ref[0]
    p1 = p1_ref[0]
    lane = lax.broadcasted_iota(jnp.int32, p0.shape, 1)
    cnt = jnp.sum(jnp.where(lane == D_HALF, p0, 0.0), axis=1, keepdims=True)
    cnt = jnp.maximum(cnt, 1.0)
    agg = jnp.concatenate(
        [p0[:, :D_HALF], p1[:, D_PAD - D_HALF:]], axis=1) / cnt
    x = jnp.maximum(agg + root_ref[...] + bias_ref[...], 0.0)
    m = jnp.max(x, axis=1, keepdims=True)
    ssum = jnp.sum(jnp.exp(x - m), axis=1, keepdims=True)
    out_ref[...] = x - (m + jnp.log(ssum))


def _finalize(p, root, bias2, nb=1000):
    nblk = N_NODES // nb
    return pl.pallas_call(
        _final_body,
        grid=(nblk,),
        in_specs=[
            pl.BlockSpec((1, nb, D_PAD), lambda n: (0, n, 0)),
            pl.BlockSpec((1, nb, D_PAD), lambda n: (1, n, 0)),
            pl.BlockSpec((nb, D_OUT), lambda n: (n, 0)),
            pl.BlockSpec((1, D_OUT), lambda n: (0, 0)),
        ],
        out_specs=pl.BlockSpec((nb, D_OUT), lambda n: (n, 0)),
        out_shape=jax.ShapeDtypeStruct((N_NODES, D_OUT), jnp.float32),
    )(p, p, root, bias2)


def kernel(edge_index, edge_type, edge_norm, comp, basis, root, bias):
    src = edge_index[0]
    dst = edge_index[1]
    et2 = edge_type.reshape(80, 2000).astype(jnp.int32)
    src2 = src.reshape(80, 2000).astype(jnp.int32)
    w3p, eidx = _build_w(comp, et2, src2, basis)
    w3 = w3p.reshape(ROW_VIEW, D_PAD)
    eidx3 = eidx.reshape(2, NSUB, NBLK_E * CH_PER_BLK, CHUNK)
    dst3 = dst.reshape(NSUB, NBLK_E * CH_PER_BLK, CHUNK).astype(jnp.int32)
    p = _sc_aggregate(eidx3, dst3, w3)
    p = p.reshape(2, N_ACC, D_PAD)
    bias2 = bias.reshape(1, D_OUT)
    return _finalize(p, root, bias2)


# speedup vs baseline: 1.1390x; 1.1390x over previous
import jax
import jax.numpy as jnp
from jax import lax
from jax.experimental import pallas as pl
from jax.experimental.pallas import tpu as pltpu
from jax.experimental.pallas import tpu_sc as plsc

N_NODES = 10000
N_EDGES = 160000
N_REL = 5
N_BASES = 5
D_OUT = 300
D_HALF = 150
D_PAD = 160
RN = N_REL * N_NODES
NSUB = 16
EDGES_PER_SUB = N_EDGES // NSUB
CHUNK = 80
N_ACC = 10112
ROWS_PER_SUB = N_ACC // NSUB


def _build_w_body(comp_ref, et_ref, src_ref, basis_ref, out_ref, eidx_ref):
    r = pl.program_id(1)
    w = comp_ref[r, 0] * basis_ref[0]
    for b in range(1, N_BASES):
        w = w + comp_ref[r, b] * basis_ref[b]
    nb = w.shape[0]
    col = lax.broadcasted_iota(jnp.int32, (nb, D_PAD), 1)
    out_ref[0] = jnp.where(col < D_HALF, w[:, :D_PAD],
                           jnp.where(col == D_HALF, 1.0, 0.0))
    out_ref[1] = w[:, D_OUT - D_PAD:]
    eidx_ref[...] = et_ref[...] * N_NODES + src_ref[...]


def _build_w(comp, et2, src2, basis, nb=2000):
    nblk = N_NODES // nb
    return pl.pallas_call(
        _build_w_body,
        grid=(nblk, N_REL),
        in_specs=[
            pl.BlockSpec(memory_space=pltpu.SMEM),
            pl.BlockSpec((16, 2000), lambda n, r: (n, 0)),
            pl.BlockSpec((16, 2000), lambda n, r: (n, 0)),
            pl.BlockSpec((N_BASES, nb, D_OUT), lambda n, r: (0, n, 0)),
        ],
        out_specs=[
            pl.BlockSpec((2, nb, D_PAD), lambda n, r: (0, r * nblk + n, 0)),
            pl.BlockSpec((16, 2000), lambda n, r: (n, 0)),
        ],
        out_shape=[
            jax.ShapeDtypeStruct((2, RN, D_PAD), jnp.float32),
            jax.ShapeDtypeStruct((80, 2000), jnp.int32),
        ],
    )(comp, et2, src2, basis)


EBLK = 2000
NBLK_E = EDGES_PER_SUB // EBLK
CH_PER_BLK = EBLK // CHUNK


def _sc_agg_body(eidx_hbm, dst_hbm, w_hbm, out_hbm,
                 idx_v, dst_v, rows_v, acc_sh, sem):
    c = lax.axis_index("c")
    s = lax.axis_index("s")

    def zrow(i, carry):
        def zcol(k, carry2):
            rows_v[i // CHUNK, i % CHUNK, pl.ds(k * 16, 16)] = (
                jnp.zeros((16,), jnp.float32))
            return carry2
        return lax.fori_loop(0, D_PAD // 16, zcol, carry)
    lax.fori_loop(0, 2 * CHUNK, zrow, 0)

    row0 = s * ROWS_PER_SUB
    nfull = ROWS_PER_SUB // CHUNK
    tail = ROWS_PER_SUB - nfull * CHUNK

    def zcopy(q, carry):
        pltpu.sync_copy(rows_v.at[0], acc_sh.at[pl.ds(row0 + q * CHUNK, CHUNK)])
        return carry
    lax.fori_loop(0, nfull, zcopy, 0)
    pltpu.sync_copy(rows_v.at[0, pl.ds(0, tail)],
                    acc_sh.at[pl.ds(row0 + nfull * CHUNK, tail)])
    plsc.subcore_barrier()

    w_half = w_hbm.at[c]

    def block_step(k, carry):
        pltpu.sync_copy(eidx_hbm.at[s, pl.ds(k * CH_PER_BLK, CH_PER_BLK)], idx_v)
        pltpu.sync_copy(dst_hbm.at[s, pl.ds(k * CH_PER_BLK, CH_PER_BLK)], dst_v)

        pltpu.async_copy(w_half.at[idx_v.at[0]], rows_v.at[0], sem)

        def chunk_step(j, carry2):
            b = lax.rem(j, 2)
            pltpu.make_async_copy(w_half.at[idx_v.at[0]], rows_v.at[b], sem).wait()
            pltpu.async_copy(w_half.at[idx_v.at[j + 1]], rows_v.at[1 - b], sem)
            pltpu.sync_copy(rows_v.at[b], acc_sh.at[dst_v.at[j]], add=True)
            return carry2
        lax.fori_loop(0, CH_PER_BLK - 1, chunk_step, 0)
        bl = lax.rem(CH_PER_BLK - 1, 2)
        pltpu.make_async_copy(w_half.at[idx_v.at[0]], rows_v.at[bl], sem).wait()
        pltpu.sync_copy(rows_v.at[bl], acc_sh.at[dst_v.at[CH_PER_BLK - 1]],
                        add=True)
        return carry
    lax.fori_loop(0, NBLK_E, block_step, 0)

    plsc.subcore_barrier()
    pltpu.sync_copy(acc_sh.at[pl.ds(row0, ROWS_PER_SUB)],
                    out_hbm.at[pl.ds(c * N_ACC + row0, ROWS_PER_SUB)])


def _sc_aggregate(eidx3, dst3, w3):
    mesh = plsc.VectorSubcoreMesh(core_axis_name="c", subcore_axis_name="s")
    f = pl.kernel(
        _sc_agg_body,
        mesh=mesh,
        compiler_params=pltpu.CompilerParams(use_tc_tiling_on_sc=False),
        out_type=jax.ShapeDtypeStruct((2 * N_ACC, D_PAD), jnp.float32),
        scratch_types=[
            pltpu.VMEM((CH_PER_BLK, CHUNK), jnp.int32),
            pltpu.VMEM((CH_PER_BLK, CHUNK), jnp.int32),
            pltpu.VMEM((2, CHUNK, D_PAD), jnp.float32),
            pltpu.VMEM_SHARED((N_ACC, D_PAD), jnp.float32),
            pltpu.SemaphoreType.DMA,
        ],
    )
    return f(eidx3, dst3, w3)


def _final_body(p0_ref, p1_ref, root_ref, bias_ref, out_ref):
    p0 = p0_ref[0]
    p1 = p1_ref[0]
    lane = lax.broadcasted_iota(jnp.int32, p0.shape, 1)
    cnt = jnp.sum(jnp.where(lane == D_HALF, p0, 0.0), axis=1, keepdims=True)
    cnt = jnp.maximum(cnt, 1.0)
    agg = jnp.concatenate(
        [p0[:, :D_HALF], p1[:, D_PAD - D_HALF:]], axis=1) / cnt
    x = jnp.maximum(agg + root_ref[...] + bias_ref[...], 0.0)
    m = jnp.max(x, axis=1, keepdims=True)
    ssum = jnp.sum(jnp.exp(x - m), axis=1, keepdims=True)
    out_ref[...] = x - (m + jnp.log(ssum))


def _finalize(p, root, bias2, nb=1000):
    nblk = N_NODES // nb
    return pl.pallas_call(
        _final_body,
        grid=(nblk,),
        in_specs=[
            pl.BlockSpec((1, nb, D_PAD), lambda n: (0, n, 0)),
            pl.BlockSpec((1, nb, D_PAD), lambda n: (1, n, 0)),
            pl.BlockSpec((nb, D_OUT), lambda n: (n, 0)),
            pl.BlockSpec((1, D_OUT), lambda n: (0, 0)),
        ],
        out_specs=pl.BlockSpec((nb, D_OUT), lambda n: (n, 0)),
        out_shape=jax.ShapeDtypeStruct((N_NODES, D_OUT), jnp.float32),
    )(p, p, root, bias2)


def kernel(edge_index, edge_type, edge_norm, comp, basis, root, bias):
    src = edge_index[0]
    dst = edge_index[1]
    et2 = edge_type.reshape(80, 2000).astype(jnp.int32)
    src2 = src.reshape(80, 2000).astype(jnp.int32)
    w3, eidx = _build_w(comp, et2, src2, basis)
    eidx3 = eidx.reshape(NSUB, NBLK_E * CH_PER_BLK, CHUNK)
    dst3 = dst.reshape(NSUB, NBLK_E * CH_PER_BLK, CHUNK).astype(jnp.int32)
    p = _sc_aggregate(eidx3, dst3, w3)
    p = p.reshape(2, N_ACC, D_PAD)
    bias2 = bias.reshape(1, D_OUT)
    return _finalize(p, root, bias2)
